# baseline (speedup 1.0000x reference)
import jax
import jax.numpy as jnp
from jax import lax
from jax.experimental import pallas as pl
from jax.experimental.pallas import tpu as pltpu
from jax.experimental.pallas import tpu_sc as plsc

N = 10000
E = 320000
D = 128
NC = 40
NP = 10240
EROWS = E // 128
RPT = 78
NEXTRA = EROWS - 32 * RPT
B = 512
GRID = NP // B
NSUB = NP // 16

_mesh = plsc.VectorSubcoreMesh(core_axis_name="c", subcore_axis_name="s")


def _fill(ref, n16, value):
    v = jnp.full((16,), value, jnp.float32)

    def st(i, carry):
        ref[pl.ds(i * 16, 16)] = v
        return carry

    lax.fori_loop(0, n16, st, 0)


def _deg_body(ei_hbm, out_hbm, dtab, dstv, ones, zv, isem0, isem1,
              ssem0, ssem1):
    c = lax.axis_index("c")
    s = lax.axis_index("s")
    r0 = s * NSUB
    _fill(ones, 8, 1.0)
    _fill(zv, 8, 0.0)
    for q in range(NSUB // 128):
        pltpu.sync_copy(zv, dtab.at[pl.ds(r0 + q * 128, 128)])
    plsc.subcore_barrier()
    w = c * 16 + s
    row0 = w * RPT
    isems = (isem0, isem1)
    ssems = (ssem0, ssem1)

    def load_idx(r, b):
        return pltpu.async_copy(
            ei_hbm.at[1, pl.ds(r * 128, 128)], dstv.at[b], isems[b])

    def fire_scatter(b):
        return pltpu.async_copy(ones, dtab.at[dstv.at[b]], ssems[b],
                                add=True)

    def wait_scatter(b):
        pltpu.make_async_copy(ones, dtab.at[dstv.at[b]], ssems[b]).wait()

    i0 = load_idx(row0, 0)
    i1 = load_idx(row0 + 1, 1)
    i0.wait()
    fire_scatter(0)
    i1.wait()
    fire_scatter(1)

    def body(k, carry):
        r = row0 + 2 * k
        wait_scatter(0)
        ia = load_idx(r, 0)
        wait_scatter(1)
        ib = load_idx(r + 1, 1)
        ia.wait()
        fire_scatter(0)
        ib.wait()
        fire_scatter(1)
        return carry

    lax.fori_loop(1, RPT // 2, body, 0)
    wait_scatter(0)
    wait_scatter(1)

    @pl.when(w < NEXTRA)
    def _():
        load_idx(32 * RPT + w, 0).wait()
        fire_scatter(0)
        wait_scatter(0)

    plsc.subcore_barrier()
    pltpu.sync_copy(dtab.at[pl.ds(r0, NSUB)], out_hbm.at[c, pl.ds(r0, NSUB)])


_deg = pl.kernel(
    _deg_body,
    out_type=jax.ShapeDtypeStruct((2, NP), jnp.float32),
    mesh=_mesh,
    scratch_types=[
        pltpu.VMEM_SHARED((NP,), jnp.float32),
        pltpu.VMEM((2, 128), jnp.int32),
        pltpu.VMEM((128,), jnp.float32),
        pltpu.VMEM((128,), jnp.float32),
        pltpu.SemaphoreType.DMA,
        pltpu.SemaphoreType.DMA,
        pltpu.SemaphoreType.DMA,
        pltpu.SemaphoreType.DMA,
    ],
)


DH = D // 2
RPT2 = 156


def _agg_body(y_hbm, ei_hbm, out_hbm,
              acc, ytab, srcv, dstv, rows0, rows1,
              gsem0, gsem1, ssem0, ssem1):
    c = lax.axis_index("c")
    s = lax.axis_index("s")
    r0 = s * NSUB
    pltpu.sync_copy(y_hbm.at[c, pl.ds(r0, NSUB)], ytab.at[pl.ds(r0, NSUB)])
    pltpu.sync_copy(y_hbm.at[c, pl.ds(r0, NSUB)], acc.at[pl.ds(r0, NSUB)])
    plsc.subcore_barrier()
    row0 = s * RPT2
    rowsb = (rows0, rows1)
    gsems = (gsem0, gsem1)
    ssems = (ssem0, ssem1)

    def load_and_gather(r, b):
        pltpu.sync_copy(ei_hbm.at[0, pl.ds(r * 128, 128)], srcv.at[b])
        pltpu.sync_copy(ei_hbm.at[1, pl.ds(r * 128, 128)], dstv.at[b])
        return pltpu.async_copy(ytab.at[srcv.at[b]], rowsb[b], gsems[b])

    def fire_scatter(b):
        return pltpu.async_copy(rowsb[b], acc.at[dstv.at[b]], ssems[b],
                                add=True)

    def wait_scatter(b):
        pltpu.make_async_copy(rowsb[b], acc.at[dstv.at[b]], ssems[b]).wait()

    g0 = load_and_gather(row0, 0)
    g1 = load_and_gather(row0 + 1, 1)
    g0.wait()
    fire_scatter(0)
    g1.wait()
    fire_scatter(1)

    def body(k, carry):
        r = row0 + 2 * k
        wait_scatter(0)
        ga = load_and_gather(r, 0)
        wait_scatter(1)
        gb = load_and_gather(r + 1, 1)
        ga.wait()
        fire_scatter(0)
        gb.wait()
        fire_scatter(1)
        return carry

    lax.fori_loop(1, RPT2 // 2, body, 0)
    wait_scatter(0)
    wait_scatter(1)

    @pl.when(s < NEXTRA)
    def _():
        g = load_and_gather(16 * RPT2 + s, 0)
        g.wait()
        fire_scatter(0)
        wait_scatter(0)

    plsc.subcore_barrier()
    pltpu.sync_copy(acc.at[pl.ds(r0, NSUB)], out_hbm.at[c, pl.ds(r0, NSUB)])


_agg = pl.kernel(
    _agg_body,
    out_type=jax.ShapeDtypeStruct((2, NP, DH), jnp.float32),
    mesh=_mesh,
    scratch_types=[
        pltpu.VMEM_SHARED((NP, DH), jnp.float32),
        pltpu.VMEM_SHARED((NP, DH), jnp.float32),
        pltpu.VMEM((2, 128), jnp.int32),
        pltpu.VMEM((2, 128), jnp.int32),
        pltpu.VMEM((128, DH), jnp.float32),
        pltpu.VMEM((128, DH), jnp.float32),
        pltpu.SemaphoreType.DMA,
        pltpu.SemaphoreType.DMA,
        pltpu.SemaphoreType.DMA,
        pltpu.SemaphoreType.DMA,
    ],
)


def _split_store(o_ref, y):
    o_ref[0] = y[:, :DH]
    o_ref[1] = y[:, DH:]


def _k1_body(x_ref, w_ref, deg_ref, o_ref, dinv_ref):
    i = pl.program_id(0)
    xw = jnp.dot(x_ref[...], w_ref[...], preferred_element_type=jnp.float32)
    d = deg_ref[0, pl.ds(i * B, B)] + deg_ref[1, pl.ds(i * B, B)] + 1.0
    dinv = lax.rsqrt(d)[:, None]
    _split_store(o_ref, xw * dinv)
    dinv_ref[...] = dinv


_k1 = pl.pallas_call(
    _k1_body,
    grid=(GRID,),
    in_specs=[
        pl.BlockSpec((B, D), lambda i: (i, 0)),
        pl.BlockSpec((D, D), lambda i: (0, 0)),
        pl.BlockSpec((2, NP), lambda i: (0, 0)),
    ],
    out_specs=[
        pl.BlockSpec((2, B, DH), lambda i: (0, i, 0)),
        pl.BlockSpec((B, 1), lambda i: (i, 0)),
    ],
    out_shape=[
        jax.ShapeDtypeStruct((2, NP, DH), jnp.float32),
        jax.ShapeDtypeStruct((NP, 1), jnp.float32),
    ],
)


def _k2_body(acc_ref, dinv_ref, b1_ref, w_ref, o_ref):
    dinv = dinv_ref[...]
    agg = jnp.concatenate([acc_ref[0], acc_ref[1]], axis=1)
    h = jnp.maximum(agg * dinv + b1_ref[...], 0.0)
    _split_store(o_ref, jnp.dot(
        h, w_ref[...], preferred_element_type=jnp.float32) * dinv)


_k2 = pl.pallas_call(
    _k2_body,
    grid=(GRID,),
    in_specs=[
        pl.BlockSpec((2, B, DH), lambda i: (0, i, 0)),
        pl.BlockSpec((B, 1), lambda i: (i, 0)),
        pl.BlockSpec((1, D), lambda i: (0, 0)),
        pl.BlockSpec((D, D), lambda i: (0, 0)),
    ],
    out_specs=pl.BlockSpec((2, B, DH), lambda i: (0, i, 0)),
    out_shape=jax.ShapeDtypeStruct((2, NP, DH), jnp.float32),
)


def _k3_body(acc_ref, dinv_ref, b2_ref, w3_ref, b3_ref, o_ref):
    dinv = dinv_ref[...]
    agg = jnp.concatenate([acc_ref[0], acc_ref[1]], axis=1)
    h = agg * dinv + b2_ref[...]
    logits = jnp.dot(h, w3_ref[...], preferred_element_type=jnp.float32)
    logits = logits + b3_ref[...]
    mask = lax.broadcasted_iota(jnp.int32, (B, D), 1) < NC
    neg = jnp.where(mask, logits, -jnp.inf)
    m = jnp.max(neg, axis=1, keepdims=True)
    e = jnp.where(mask, jnp.exp(logits - m), 0.0)
    lse = m + jnp.log(jnp.sum(e, axis=1, keepdims=True))
    o_ref[...] = (logits - lse)[:, :NC]


_k3 = pl.pallas_call(
    _k3_body,
    grid=(GRID,),
    in_specs=[
        pl.BlockSpec((2, B, DH), lambda i: (0, i, 0)),
        pl.BlockSpec((B, 1), lambda i: (i, 0)),
        pl.BlockSpec((1, D), lambda i: (0, 0)),
        pl.BlockSpec((D, D), lambda i: (0, 0)),
        pl.BlockSpec((1, D), lambda i: (0, 0)),
    ],
    out_specs=pl.BlockSpec((B, NC), lambda i: (i, 0)),
    out_shape=jax.ShapeDtypeStruct((N, NC), jnp.float32),
)


def kernel(x, edge_index, W1, b1, W2, b2, W3, b3):
    ei = edge_index.astype(jnp.int32)
    b1r = b1.reshape(1, D)
    b2r = b2.reshape(1, D)
    W3p = jnp.pad(W3, ((0, 0), (0, D - NC)))
    b3r = jnp.pad(b3, (0, D - NC)).reshape(1, D)

    deg_pair = _deg(ei)
    y1, dinvc = _k1(x, W1, deg_pair)
    acc1 = _agg(y1, ei)
    y2 = _k2(acc1, dinvc, b1r, W2)
    acc2 = _agg(y2, ei)
    return _k3(acc2, dinvc, b2r, W3p, b3r)


# speedup vs baseline: 1.4858x; 1.4858x over previous
import jax
import jax.numpy as jnp
from jax import lax
from jax.experimental import pallas as pl
from jax.experimental.pallas import tpu as pltpu
from jax.experimental.pallas import tpu_sc as plsc

N = 10000
E = 320000
D = 128
NC = 40
NP = 10240
EROWS = E // 128
RPT = 78
NEXTRA = EROWS - 32 * RPT
B = 1024
GRID = NP // B
NSUB = NP // 16

_mesh = plsc.VectorSubcoreMesh(core_axis_name="c", subcore_axis_name="s")


def _fill(ref, n16, value):
    v = jnp.full((16,), value, jnp.float32)

    def st(i, carry):
        ref[pl.ds(i * 16, 16)] = v
        return carry

    lax.fori_loop(0, n16, st, 0)


def _deg_body(ei_hbm, out_hbm, dtab, dstv, ones, zv,
              isem0, isem1, isem2, isem3, ssem0, ssem1, ssem2, ssem3):
    c = lax.axis_index("c")
    s = lax.axis_index("s")
    r0 = s * NSUB
    _fill(ones, 8, 1.0)
    _fill(zv, 8, 0.0)
    for q in range(NSUB // 128):
        pltpu.sync_copy(zv, dtab.at[pl.ds(r0 + q * 128, 128)])
    plsc.subcore_barrier()
    w = c * 16 + s
    row0 = w * RPT
    isems = (isem0, isem1, isem2, isem3)
    ssems = (ssem0, ssem1, ssem2, ssem3)

    def load_idx(r, b):
        return pltpu.async_copy(
            ei_hbm.at[1, pl.ds(r * 128, 128)], dstv.at[b], isems[b])

    def fire_scatter(b):
        return pltpu.async_copy(ones, dtab.at[dstv.at[b]], ssems[b],
                                add=True)

    def wait_scatter(b):
        pltpu.make_async_copy(ones, dtab.at[dstv.at[b]], ssems[b]).wait()

    pre = [load_idx(row0 + b, b) for b in range(4)]
    for b in range(4):
        pre[b].wait()
        fire_scatter(b)

    def body(k, carry):
        r = row0 + 4 * k
        loads = []
        for b in range(4):
            wait_scatter(b)
            loads.append(load_idx(r + b, b))
        for b in range(4):
            loads[b].wait()
            fire_scatter(b)
        return carry

    lax.fori_loop(1, RPT // 4, body, 0)
    for b in range(RPT % 4):
        wait_scatter(b)
        load_idx(row0 + 4 * (RPT // 4) + b, b).wait()
        fire_scatter(b)

    @pl.when(w < NEXTRA)
    def _():
        wait_scatter(3)
        load_idx(32 * RPT + w, 3).wait()
        fire_scatter(3)

    for b in range(4):
        wait_scatter(b)

    plsc.subcore_barrier()
    pltpu.sync_copy(dtab.at[pl.ds(r0, NSUB)], out_hbm.at[c, pl.ds(r0, NSUB)])


_deg = pl.kernel(
    _deg_body,
    out_type=jax.ShapeDtypeStruct((2, NP), jnp.float32),
    mesh=_mesh,
    scratch_types=[
        pltpu.VMEM_SHARED((NP,), jnp.float32),
        pltpu.VMEM((4, 128), jnp.int32),
        pltpu.VMEM((128,), jnp.float32),
        pltpu.VMEM((128,), jnp.float32),
    ] + [pltpu.SemaphoreType.DMA] * 8,
)


def _agg_body(y_hbm, ei_hbm, out_hbm,
              acc, srcv, dstv, rows0, rows1, gsem0, gsem1, ssem0, ssem1):
    c = lax.axis_index("c")
    s = lax.axis_index("s")
    r0 = s * NSUB

    @pl.when(c == 0)
    def _():
        pltpu.sync_copy(y_hbm.at[pl.ds(r0, NSUB)], acc.at[pl.ds(r0, NSUB)])

    @pl.when(c != 0)
    def _():
        def zall(t, carry):
            rows0[t // 8, pl.ds((t % 8) * 16, 16)] = jnp.zeros(
                (16,), jnp.float32)
            return carry

        lax.fori_loop(0, 1024, zall, 0)
        for q in range(NSUB // 128):
            pltpu.sync_copy(rows0, acc.at[pl.ds(r0 + q * 128, 128)])

    plsc.subcore_barrier()
    w = c * 16 + s
    row0 = w * RPT
    rowsb = (rows0, rows1)
    gsems = (gsem0, gsem1)
    ssems = (ssem0, ssem1)

    def load_and_gather(r, b):
        pltpu.sync_copy(ei_hbm.at[0, pl.ds(r * 128, 128)], srcv.at[b])
        pltpu.sync_copy(ei_hbm.at[1, pl.ds(r * 128, 128)], dstv.at[b])
        return pltpu.async_copy(y_hbm.at[srcv.at[b]], rowsb[b], gsems[b])

    def fire_scatter(b):
        return pltpu.async_copy(rowsb[b], acc.at[dstv.at[b]], ssems[b],
                                add=True)

    def wait_scatter(b):
        pltpu.make_async_copy(rowsb[b], acc.at[dstv.at[b]], ssems[b]).wait()

    g0 = load_and_gather(row0, 0)
    g1 = load_and_gather(row0 + 1, 1)
    g0.wait()
    fire_scatter(0)
    g1.wait()
    fire_scatter(1)

    def body(k, carry):
        r = row0 + 2 * k
        wait_scatter(0)
        ga = load_and_gather(r, 0)
        wait_scatter(1)
        gb = load_and_gather(r + 1, 1)
        ga.wait()
        fire_scatter(0)
        gb.wait()
        fire_scatter(1)
        return carry

    lax.fori_loop(1, RPT // 2, body, 0)
    wait_scatter(0)
    wait_scatter(1)

    @pl.when(w < NEXTRA)
    def _():
        g = load_and_gather(32 * RPT + w, 0)
        g.wait()
        fire_scatter(0)
        wait_scatter(0)

    plsc.subcore_barrier()
    pltpu.sync_copy(acc.at[pl.ds(r0, NSUB)], out_hbm.at[c, pl.ds(r0, NSUB)])


_agg = pl.kernel(
    _agg_body,
    out_type=jax.ShapeDtypeStruct((2, NP, D), jnp.float32),
    mesh=_mesh,
    scratch_types=[
        pltpu.VMEM_SHARED((NP, D), jnp.float32),
        pltpu.VMEM((2, 128), jnp.int32),
        pltpu.VMEM((2, 128), jnp.int32),
        pltpu.VMEM((128, D), jnp.float32),
        pltpu.VMEM((128, D), jnp.float32),
        pltpu.SemaphoreType.DMA,
        pltpu.SemaphoreType.DMA,
        pltpu.SemaphoreType.DMA,
        pltpu.SemaphoreType.DMA,
    ],
)


def _k1_body(x_ref, w_ref, deg_ref, o_ref, dinv_ref):
    i = pl.program_id(0)
    xw = jnp.dot(x_ref[...], w_ref[...], preferred_element_type=jnp.float32)
    d = deg_ref[0, pl.ds(i * B, B)] + deg_ref[1, pl.ds(i * B, B)] + 1.0
    dinv = lax.rsqrt(d)[:, None]
    o_ref[...] = xw * dinv
    dinv_ref[...] = dinv


_k1 = pl.pallas_call(
    _k1_body,
    grid=(GRID,),
    in_specs=[
        pl.BlockSpec((B, D), lambda i: (i, 0)),
        pl.BlockSpec((D, D), lambda i: (0, 0)),
        pl.BlockSpec((2, NP), lambda i: (0, 0)),
    ],
    out_specs=[
        pl.BlockSpec((B, D), lambda i: (i, 0)),
        pl.BlockSpec((B, 1), lambda i: (i, 0)),
    ],
    out_shape=[
        jax.ShapeDtypeStruct((NP, D), jnp.float32),
        jax.ShapeDtypeStruct((NP, 1), jnp.float32),
    ],
)


def _k2_body(acc_ref, dinv_ref, b1_ref, w_ref, o_ref):
    dinv = dinv_ref[...]
    h = jnp.maximum((acc_ref[0] + acc_ref[1]) * dinv + b1_ref[...], 0.0)
    o_ref[...] = jnp.dot(
        h, w_ref[...], preferred_element_type=jnp.float32) * dinv


_k2 = pl.pallas_call(
    _k2_body,
    grid=(GRID,),
    in_specs=[
        pl.BlockSpec((2, B, D), lambda i: (0, i, 0)),
        pl.BlockSpec((B, 1), lambda i: (i, 0)),
        pl.BlockSpec((1, D), lambda i: (0, 0)),
        pl.BlockSpec((D, D), lambda i: (0, 0)),
    ],
    out_specs=pl.BlockSpec((B, D), lambda i: (i, 0)),
    out_shape=jax.ShapeDtypeStruct((NP, D), jnp.float32),
)


def _k3_body(acc_ref, dinv_ref, b2_ref, w3_ref, b3_ref, o_ref):
    dinv = dinv_ref[...]
    h = (acc_ref[0] + acc_ref[1]) * dinv + b2_ref[...]
    logits = jnp.dot(h, w3_ref[...], preferred_element_type=jnp.float32)
    logits = logits + b3_ref[...]
    mask = lax.broadcasted_iota(jnp.int32, (B, D), 1) < NC
    neg = jnp.where(mask, logits, -jnp.inf)
    m = jnp.max(neg, axis=1, keepdims=True)
    e = jnp.where(mask, jnp.exp(logits - m), 0.0)
    lse = m + jnp.log(jnp.sum(e, axis=1, keepdims=True))
    o_ref[...] = (logits - lse)[:, :NC]


_k3 = pl.pallas_call(
    _k3_body,
    grid=(GRID,),
    in_specs=[
        pl.BlockSpec((2, B, D), lambda i: (0, i, 0)),
        pl.BlockSpec((B, 1), lambda i: (i, 0)),
        pl.BlockSpec((1, D), lambda i: (0, 0)),
        pl.BlockSpec((D, D), lambda i: (0, 0)),
        pl.BlockSpec((1, D), lambda i: (0, 0)),
    ],
    out_specs=pl.BlockSpec((B, NC), lambda i: (i, 0)),
    out_shape=jax.ShapeDtypeStruct((N, NC), jnp.float32),
)


def kernel(x, edge_index, W1, b1, W2, b2, W3, b3):
    ei = edge_index.astype(jnp.int32)
    b1r = b1.reshape(1, D)
    b2r = b2.reshape(1, D)
    W3p = jnp.pad(W3, ((0, 0), (0, D - NC)))
    b3r = jnp.pad(b3, (0, D - NC)).reshape(1, D)

    deg_pair = _deg(ei)
    y1, dinvc = _k1(x, W1, deg_pair)
    acc1 = _agg(y1, ei)
    y2 = _k2(acc1, dinvc, b1r, W2)
    acc2 = _agg(y2, ei)
    return _k3(acc2, dinvc, b2r, W3p, b3r)
